# initial kernel scaffold (unmeasured)
import jax
import jax.numpy as jnp
from jax import lax
from jax.experimental import pallas as pl
from jax.experimental.pallas import tpu as pltpu

N_DEV = 4


def _ring_allreduce(partial):
    n, d = partial.shape
    chunk = n // N_DEV

    def body(p_ref, out_ref, comm_ref, send_sems, recv_sems):
        my = lax.axis_index("i")
        left = (my - 1) % N_DEV
        right = (my + 1) % N_DEV

        barrier_sem = pltpu.get_barrier_semaphore()
        for nbr in (left, right):
            pl.semaphore_signal(
                barrier_sem, inc=1,
                device_id=(nbr,), device_id_type=pl.DeviceIdType.MESH,
            )
        pl.semaphore_wait(barrier_sem, 2)

        out_ref[:, :] = p_ref[:, :]

        for s in range(N_DEV - 1):
            send_off = ((my - s) % N_DEV) * chunk
            rdma = pltpu.make_async_remote_copy(
                src_ref=out_ref.at[pl.ds(send_off, chunk), :],
                dst_ref=comm_ref.at[s],
                send_sem=send_sems.at[s],
                recv_sem=recv_sems.at[s],
                device_id=(right,),
                device_id_type=pl.DeviceIdType.MESH,
            )
            rdma.start()
            rdma.wait()
            recv_off = ((my - 1 - s) % N_DEV) * chunk
            out_ref[pl.ds(recv_off, chunk), :] = (
                out_ref[pl.ds(recv_off, chunk), :] + comm_ref[s, :, :]
            )

        for t in range(N_DEV - 1):
            k = (N_DEV - 1) + t
            send_off = ((my + 1 - t) % N_DEV) * chunk
            rdma = pltpu.make_async_remote_copy(
                src_ref=out_ref.at[pl.ds(send_off, chunk), :],
                dst_ref=comm_ref.at[k],
                send_sem=send_sems.at[k],
                recv_sem=recv_sems.at[k],
                device_id=(right,),
                device_id_type=pl.DeviceIdType.MESH,
            )
            rdma.start()
            rdma.wait()
            recv_off = ((my - t) % N_DEV) * chunk
            out_ref[pl.ds(recv_off, chunk), :] = comm_ref[k, :, :]

    n_steps = 2 * (N_DEV - 1)
    return pl.pallas_call(
        body,
        out_shape=jax.ShapeDtypeStruct((n, d), partial.dtype),
        in_specs=[pl.BlockSpec(memory_space=pltpu.VMEM)],
        out_specs=pl.BlockSpec(memory_space=pltpu.VMEM),
        scratch_shapes=[
            pltpu.VMEM((n_steps, chunk, d), partial.dtype),
            pltpu.SemaphoreType.DMA((n_steps,)),
            pltpu.SemaphoreType.DMA((n_steps,)),
        ],
        compiler_params=pltpu.CompilerParams(collective_id=0),
    )(partial)


def kernel(table, idx):
    v_per = table.shape[0]
    my = lax.axis_index("i")
    local = idx.astype(jnp.int32) - my * v_per
    owned = (local >= 0) & (local < v_per)
    safe = jnp.where(owned, local, 0)
    partial = jnp.where(owned[:, None], table[safe], jnp.float32(0.0))
    return _ring_allreduce(partial)


# baseline (device time: 208533 ns/iter reference)
import jax
import jax.numpy as jnp
from jax import lax
from jax.experimental import pallas as pl
from jax.experimental.pallas import tpu as pltpu

N_DEV = 4


def kernel(table, idx):
    v_per, d = table.shape
    n = idx.shape[0]
    chunk = n // N_DEV
    n_steps = 2 * (N_DEV - 1)

    def body(table_ref, idx_ref, out_ref, comm_ref, gather_sem,
             send_sems, recv_sems):
        my = lax.axis_index("i")
        left = (my - 1) % N_DEV
        right = (my + 1) % N_DEV

        out_ref[:, :] = jnp.zeros((n, d), jnp.float32)

        def issue(j, cnt):
            row = idx_ref[j] - my * v_per
            owned = (row >= 0) & (row < v_per)
            safe = jnp.clip(row, 0, v_per - 1)

            @pl.when(owned)
            def _():
                pltpu.make_async_copy(
                    table_ref.at[pl.ds(safe, 1), :],
                    out_ref.at[pl.ds(j, 1), :],
                    gather_sem,
                ).start()

            return cnt + owned.astype(jnp.int32)

        n_owned = lax.fori_loop(0, n, issue, jnp.int32(0))

        barrier_sem = pltpu.get_barrier_semaphore()
        for nbr in (left, right):
            pl.semaphore_signal(
                barrier_sem, inc=1,
                device_id=(nbr,), device_id_type=pl.DeviceIdType.MESH,
            )
        pl.semaphore_wait(barrier_sem, 2)

        def drain(j, _):
            pltpu.make_async_copy(
                table_ref.at[pl.ds(0, 1), :],
                out_ref.at[pl.ds(0, 1), :],
                gather_sem,
            ).wait()
            return _

        lax.fori_loop(0, n_owned, drain, jnp.int32(0))

        for s in range(N_DEV - 1):
            send_off = ((my - s) % N_DEV) * chunk
            rdma = pltpu.make_async_remote_copy(
                src_ref=out_ref.at[pl.ds(send_off, chunk), :],
                dst_ref=comm_ref.at[s],
                send_sem=send_sems.at[s],
                recv_sem=recv_sems.at[s],
                device_id=(right,),
                device_id_type=pl.DeviceIdType.MESH,
            )
            rdma.start()
            rdma.wait()
            recv_off = ((my - 1 - s) % N_DEV) * chunk
            out_ref[pl.ds(recv_off, chunk), :] = (
                out_ref[pl.ds(recv_off, chunk), :] + comm_ref[s, :, :]
            )

        for t in range(N_DEV - 1):
            k = (N_DEV - 1) + t
            send_off = ((my + 1 - t) % N_DEV) * chunk
            rdma = pltpu.make_async_remote_copy(
                src_ref=out_ref.at[pl.ds(send_off, chunk), :],
                dst_ref=comm_ref.at[k],
                send_sem=send_sems.at[k],
                recv_sem=recv_sems.at[k],
                device_id=(right,),
                device_id_type=pl.DeviceIdType.MESH,
            )
            rdma.start()
            rdma.wait()
            recv_off = ((my - t) % N_DEV) * chunk
            out_ref[pl.ds(recv_off, chunk), :] = comm_ref[k, :, :]

    return pl.pallas_call(
        body,
        out_shape=jax.ShapeDtypeStruct((n, d), jnp.float32),
        in_specs=[
            pl.BlockSpec(memory_space=pltpu.MemorySpace.HBM),
            pl.BlockSpec(memory_space=pltpu.SMEM),
        ],
        out_specs=pl.BlockSpec(memory_space=pltpu.VMEM),
        scratch_shapes=[
            pltpu.VMEM((n_steps, chunk, d), jnp.float32),
            pltpu.SemaphoreType.DMA,
            pltpu.SemaphoreType.DMA((n_steps,)),
            pltpu.SemaphoreType.DMA((n_steps,)),
        ],
        compiler_params=pltpu.CompilerParams(collective_id=0),
    )(table, idx.astype(jnp.int32))


# device time: 143547 ns/iter; 1.4527x vs baseline; 1.4527x over previous
import jax
import jax.numpy as jnp
from jax import lax
from jax.experimental import pallas as pl
from jax.experimental.pallas import tpu as pltpu

N_DEV = 4


def kernel(table, idx):
    v_per, d = table.shape
    n = idx.shape[0]
    half = n // 2
    chunk = half // N_DEV
    n_steps = 2 * (N_DEV - 1)

    def body(table_ref, idx_ref, out_ref, comm_cw, comm_ccw, counts_ref,
             gather_sems, send_cw, recv_cw, send_ccw, recv_ccw):
        my = lax.axis_index("i")
        left = (my - 1) % N_DEV
        right = (my + 1) % N_DEV

        def cw_rows(c):
            return pl.ds((c % N_DEV) * chunk, chunk)

        def ccw_rows(c):
            return pl.ds(half + (c % N_DEV) * chunk, chunk)

        out_ref[:, :] = jnp.zeros((n, d), jnp.float32)
        for c in range(2 * N_DEV):
            counts_ref[c] = 0

        def issue(j, carry):
            row = idx_ref[j] - my * v_per
            owned = (row >= 0) & (row < v_per)
            safe = jnp.clip(row, 0, v_per - 1)
            c = j // chunk

            @pl.when(owned)
            def _():
                pltpu.make_async_copy(
                    table_ref.at[pl.ds(safe, 1), :],
                    out_ref.at[pl.ds(j, 1), :],
                    gather_sems.at[c],
                ).start()
                counts_ref[c] = counts_ref[c] + 1

            return carry

        lax.fori_loop(0, n, issue, jnp.int32(0))

        def drain(c):
            def w(i, carry):
                pltpu.make_async_copy(
                    table_ref.at[pl.ds(0, 1), :],
                    out_ref.at[pl.ds(0, 1), :],
                    gather_sems.at[c],
                ).wait()
                return carry

            lax.fori_loop(0, counts_ref[c], w, jnp.int32(0))

        barrier_sem = pltpu.get_barrier_semaphore()
        for nbr in (left, right):
            pl.semaphore_signal(
                barrier_sem, inc=1,
                device_id=(nbr,), device_id_type=pl.DeviceIdType.MESH,
            )
        pl.semaphore_wait(barrier_sem, 2)

        drain((my % N_DEV))
        drain(N_DEV + (my % N_DEV))
        for s in range(N_DEV - 1):
            rdma_cw = pltpu.make_async_remote_copy(
                src_ref=out_ref.at[cw_rows(my - s), :],
                dst_ref=comm_cw.at[s],
                send_sem=send_cw.at[s],
                recv_sem=recv_cw.at[s],
                device_id=(right,),
                device_id_type=pl.DeviceIdType.MESH,
            )
            rdma_cw.start()
            rdma_ccw = pltpu.make_async_remote_copy(
                src_ref=out_ref.at[ccw_rows(my + s), :],
                dst_ref=comm_ccw.at[s],
                send_sem=send_ccw.at[s],
                recv_sem=recv_ccw.at[s],
                device_id=(left,),
                device_id_type=pl.DeviceIdType.MESH,
            )
            rdma_ccw.start()
            drain((my - 1 - s) % N_DEV)
            drain(N_DEV + (my + 1 + s) % N_DEV)
            rdma_cw.wait()
            out_ref[cw_rows(my - 1 - s), :] = (
                out_ref[cw_rows(my - 1 - s), :] + comm_cw[s, :, :]
            )
            rdma_ccw.wait()
            out_ref[ccw_rows(my + 1 + s), :] = (
                out_ref[ccw_rows(my + 1 + s), :] + comm_ccw[s, :, :]
            )

        for t in range(N_DEV - 1):
            k = (N_DEV - 1) + t
            src_cw = (
                out_ref.at[cw_rows(my + 1), :] if t == 0
                else comm_cw.at[k - 1]
            )
            rdma_cw = pltpu.make_async_remote_copy(
                src_ref=src_cw,
                dst_ref=comm_cw.at[k],
                send_sem=send_cw.at[k],
                recv_sem=recv_cw.at[k],
                device_id=(right,),
                device_id_type=pl.DeviceIdType.MESH,
            )
            rdma_cw.start()
            src_ccw = (
                out_ref.at[ccw_rows(my - 1), :] if t == 0
                else comm_ccw.at[k - 1]
            )
            rdma_ccw = pltpu.make_async_remote_copy(
                src_ref=src_ccw,
                dst_ref=comm_ccw.at[k],
                send_sem=send_ccw.at[k],
                recv_sem=recv_ccw.at[k],
                device_id=(left,),
                device_id_type=pl.DeviceIdType.MESH,
            )
            rdma_ccw.start()
            if t >= 1:
                out_ref[cw_rows(my - (t - 1)), :] = comm_cw[k - 1, :, :]
                out_ref[ccw_rows(my + (t - 1)), :] = comm_ccw[k - 1, :, :]
            rdma_cw.wait()
            rdma_ccw.wait()
        out_ref[cw_rows(my - 2), :] = comm_cw[n_steps - 1, :, :]
        out_ref[ccw_rows(my + 2), :] = comm_ccw[n_steps - 1, :, :]

    return pl.pallas_call(
        body,
        out_shape=jax.ShapeDtypeStruct((n, d), jnp.float32),
        in_specs=[
            pl.BlockSpec(memory_space=pltpu.MemorySpace.HBM),
            pl.BlockSpec(memory_space=pltpu.SMEM),
        ],
        out_specs=pl.BlockSpec(memory_space=pltpu.VMEM),
        scratch_shapes=[
            pltpu.VMEM((n_steps, chunk, d), jnp.float32),
            pltpu.VMEM((n_steps, chunk, d), jnp.float32),
            pltpu.SMEM((2 * N_DEV,), jnp.int32),
            pltpu.SemaphoreType.DMA((2 * N_DEV,)),
            pltpu.SemaphoreType.DMA((n_steps,)),
            pltpu.SemaphoreType.DMA((n_steps,)),
            pltpu.SemaphoreType.DMA((n_steps,)),
            pltpu.SemaphoreType.DMA((n_steps,)),
        ],
        compiler_params=pltpu.CompilerParams(collective_id=0),
    )(table, idx.astype(jnp.int32))


# device time: 87813 ns/iter; 2.3747x vs baseline; 1.6347x over previous
import jax
import jax.numpy as jnp
from jax import lax
from jax.experimental import pallas as pl
from jax.experimental.pallas import tpu as pltpu

N_DEV = 4


def kernel(table, idx):
    v_per, d = table.shape
    n = idx.shape[0]
    half = n // 2
    chunk = half // N_DEV
    sub = chunk // 2
    n_steps = 2 * (N_DEV - 1)
    n_ranks = 2 * N_DEV
    n_gsems = n_ranks + 2

    def body(table_ref, idx_ref, out_ref, comm_cw, comm_ccw, gather_sems,
             send_cw, recv_cw, send_ccw, recv_ccw):
        my = lax.axis_index("i")
        left = (my - 1) % N_DEV
        right = (my + 1) % N_DEV

        def cw_base(c):
            return (c % N_DEV) * chunk

        def ccw_base(c):
            return half + (c % N_DEV) * chunk

        def seg_base(r):
            q = r // 2
            if r % 2 == 0:
                return cw_base(my - q)
            return ccw_base(my + q)

        def zero_seg(r):
            out_ref[pl.ds(seg_base(r), chunk), :] = jnp.zeros(
                (chunk, d), jnp.float32
            )

        def issue(base, count, sem_i):
            unroll = 4

            def it(i, cnt):
                j0 = base + i * unroll
                for u in range(unroll):
                    j = j0 + u
                    row = idx_ref[j] - my * v_per
                    owned = row.astype(jnp.uint32) < jnp.uint32(v_per)
                    safe = row & (v_per - 1)

                    @pl.when(owned)
                    def _():
                        pltpu.make_async_copy(
                            table_ref.at[pl.ds(safe, 1), :],
                            out_ref.at[pl.ds(j, 1), :],
                            gather_sems.at[sem_i],
                        ).start()

                    cnt = cnt + owned.astype(jnp.int32)
                return cnt

            return lax.fori_loop(0, count // unroll, it, jnp.int32(0))

        def drain(sem_i, cnt):
            def w(i, carry):
                pltpu.make_async_copy(
                    table_ref.at[pl.ds(0, 1), :],
                    out_ref.at[pl.ds(0, 1), :],
                    gather_sems.at[sem_i],
                ).wait()
                return carry

            lax.fori_loop(0, cnt, w, jnp.int32(0))

        def rs_rdma(s, b, cw):
            if cw:
                src0 = cw_base(my - s)
                comm, ssem, rsem, dev = comm_cw, send_cw, recv_cw, right
            else:
                src0 = ccw_base(my + s)
                comm, ssem, rsem, dev = comm_ccw, send_ccw, recv_ccw, left
            return pltpu.make_async_remote_copy(
                src_ref=out_ref.at[pl.ds(src0 + b * sub, sub), :],
                dst_ref=comm.at[s, pl.ds(b * sub, sub), :],
                send_sem=ssem.at[s, b],
                recv_sem=rsem.at[s, b],
                device_id=(dev,),
                device_id_type=pl.DeviceIdType.MESH,
            )

        def ag_rdma(t, b, cw):
            k = (N_DEV - 1) + t
            if cw:
                own0 = cw_base(my + 1)
                comm, ssem, rsem, dev = comm_cw, send_cw, recv_cw, right
            else:
                own0 = ccw_base(my - 1)
                comm, ssem, rsem, dev = comm_ccw, send_ccw, recv_ccw, left
            src = (
                out_ref.at[pl.ds(own0 + b * sub, sub), :] if t == 0
                else comm.at[k - 1, pl.ds(b * sub, sub), :]
            )
            return pltpu.make_async_remote_copy(
                src_ref=src,
                dst_ref=comm.at[k, pl.ds(b * sub, sub), :],
                send_sem=ssem.at[k, b],
                recv_sem=rsem.at[k, b],
                device_id=(dev,),
                device_id_type=pl.DeviceIdType.MESH,
            )

        barrier_sem = pltpu.get_barrier_semaphore()
        for nbr in (left, right):
            pl.semaphore_signal(
                barrier_sem, inc=1,
                device_id=(nbr,), device_id_type=pl.DeviceIdType.MESH,
            )

        counts = {}
        zero_seg(0)
        zero_seg(1)
        c0a = issue(seg_base(0), sub, 0)
        c1a = issue(seg_base(1), sub, 1)
        pl.semaphore_wait(barrier_sem, 2)

        drain(0, c0a)
        rs_rdma(0, 0, True).start()
        drain(1, c1a)
        rs_rdma(0, 0, False).start()
        c0b = issue(seg_base(0) + sub, sub, n_ranks)
        c1b = issue(seg_base(1) + sub, sub, n_ranks + 1)
        drain(n_ranks, c0b)
        rs_rdma(0, 1, True).start()
        drain(n_ranks + 1, c1b)
        rs_rdma(0, 1, False).start()

        for s in range(N_DEV - 1):
            zero_seg(2 * s + 2)
            counts[2 * s + 2] = issue(seg_base(2 * s + 2), chunk, 2 * s + 2)
            zero_seg(2 * s + 3)
            counts[2 * s + 3] = issue(seg_base(2 * s + 3), chunk, 2 * s + 3)
            drain(2 * s + 2, counts[2 * s + 2])
            drain(2 * s + 3, counts[2 * s + 3])
            acc_cw = cw_base(my - 1 - s)
            acc_ccw = ccw_base(my + 1 + s)
            for b in range(2):
                row_cw = pl.ds(acc_cw + b * sub, sub)
                rs_rdma(s, b, True).wait()
                out_ref[row_cw, :] = out_ref[row_cw, :] + comm_cw[
                    s, pl.ds(b * sub, sub), :
                ]
                if s < N_DEV - 2:
                    rs_rdma(s + 1, b, True).start()
                row_ccw = pl.ds(acc_ccw + b * sub, sub)
                rs_rdma(s, b, False).wait()
                out_ref[row_ccw, :] = out_ref[row_ccw, :] + comm_ccw[
                    s, pl.ds(b * sub, sub), :
                ]
                if s < N_DEV - 2:
                    rs_rdma(s + 1, b, False).start()

        for b in range(2):
            ag_rdma(0, b, True).start()
            ag_rdma(0, b, False).start()

        for t in range(N_DEV - 1):
            k = (N_DEV - 1) + t
            st_cw = cw_base(my - t)
            st_ccw = ccw_base(my + t)
            for b in range(2):
                ag_rdma(t, b, True).wait()
                if t < N_DEV - 2:
                    ag_rdma(t + 1, b, True).start()
                out_ref[pl.ds(st_cw + b * sub, sub), :] = comm_cw[
                    k, pl.ds(b * sub, sub), :
                ]
                ag_rdma(t, b, False).wait()
                if t < N_DEV - 2:
                    ag_rdma(t + 1, b, False).start()
                out_ref[pl.ds(st_ccw + b * sub, sub), :] = comm_ccw[
                    k, pl.ds(b * sub, sub), :
                ]

    return pl.pallas_call(
        body,
        out_shape=jax.ShapeDtypeStruct((n, d), jnp.float32),
        in_specs=[
            pl.BlockSpec(memory_space=pltpu.MemorySpace.HBM),
            pl.BlockSpec(memory_space=pltpu.SMEM),
        ],
        out_specs=pl.BlockSpec(memory_space=pltpu.VMEM),
        scratch_shapes=[
            pltpu.VMEM((n_steps, chunk, d), jnp.float32),
            pltpu.VMEM((n_steps, chunk, d), jnp.float32),
            pltpu.SemaphoreType.DMA((n_gsems,)),
            pltpu.SemaphoreType.DMA((n_steps, 2)),
            pltpu.SemaphoreType.DMA((n_steps, 2)),
            pltpu.SemaphoreType.DMA((n_steps, 2)),
            pltpu.SemaphoreType.DMA((n_steps, 2)),
        ],
        compiler_params=pltpu.CompilerParams(collective_id=0),
    )(table, idx.astype(jnp.int32))
